# baseline (speedup 1.0000x reference)
import functools

import jax
import jax.numpy as jnp
from jax import lax
from jax.experimental import pallas as pl
from jax.experimental.pallas import tpu as pltpu
from jax.experimental.pallas import tpu_sc as plsc

E = 8
D = 768
H = 2048
T = 2048
COEF = 0.01

GBLK = 512
NGB = T // GBLK
BLK = 256
MAX_TILES = T // BLK + E - 1
PADDED = MAX_TILES * BLK
NC, NS = 2, 16
NW = NC * NS
CHUNK = T // NW


def _gate_body(x_ref, gw_ref, gb_ref, pos_ref, te_ref, tv_ref, loss_ref,
               carry_ref, choice_s, poswi_s):
    i = pl.program_id(0)

    @pl.when(i == 0)
    def _():
        carry_ref[...] = jnp.zeros((1, E), jnp.float32)

    @pl.when(i < NGB)
    def _():
        xb = x_ref[0]
        logits = jnp.dot(xb, gw_ref[...],
                         preferred_element_type=jnp.float32) + gb_ref[...]
        lane = lax.broadcasted_iota(jnp.int32, (GBLK, E), 1)
        mx = jnp.max(logits, axis=1, keepdims=True)
        ex = jnp.exp(logits - mx)
        probs = ex / jnp.sum(ex, axis=1, keepdims=True)
        pmax = jnp.max(probs, axis=1, keepdims=True)
        first = jnp.where(probs == pmax, lane, E)
        choice = jnp.min(first, axis=1)

        onehot = (lane == choice[:, None]).astype(jnp.float32)
        r = lax.broadcasted_iota(jnp.int32, (GBLK, GBLK), 0)
        c = lax.broadcasted_iota(jnp.int32, (GBLK, GBLK), 1)
        tri = (c < r).astype(jnp.float32)
        rank = jnp.dot(tri, onehot,
                       preferred_element_type=jnp.float32) + carry_ref[...]
        poswi = jnp.sum(onehot * rank, axis=1)

        choice_s[pl.ds(i, 1), :] = choice.reshape(1, GBLK)
        poswi_s[pl.ds(i, 1), :] = poswi.astype(jnp.int32).reshape(1, GBLK)
        carry_ref[...] = carry_ref[...] + jnp.sum(onehot, axis=0,
                                                  keepdims=True)

    @pl.when(i == NGB)
    def _():
        totals = carry_ref[...]
        p = totals / float(T)
        loss = -jnp.sum(p * jnp.log(p + 1e-10)) * COEF
        loss_ref[...] = jnp.full((1, E), loss, jnp.float32)
        ch = choice_s[...]
        pw = poswi_s[...]
        jb = lax.broadcasted_iota(jnp.int32, (1, 128), 1) * BLK
        acc = jnp.zeros((NGB, GBLK), jnp.int32)
        te = jnp.zeros((1, 128), jnp.int32)
        off = jnp.int32(0)
        for e in range(E):
            te = te + (off <= jb).astype(jnp.int32)
            acc = jnp.where(ch == e, off + pw, acc)
            cnt = totals[0, e].astype(jnp.int32)
            off = off + ((cnt + BLK - 1) // BLK) * BLK
        pos_ref[...] = acc.reshape(1, T)
        te_ref[...] = te - 1
        tv_ref[...] = (jb < off).astype(jnp.int32)


def _gate_call(xf, gate_W, gate_b):
    return pl.pallas_call(
        _gate_body,
        grid=(NGB + 1,),
        in_specs=[
            pl.BlockSpec((1, GBLK, D),
                         lambda i: (0, jnp.minimum(i, NGB - 1), 0)),
            pl.BlockSpec((D, E), lambda i: (0, 0)),
            pl.BlockSpec((1, E), lambda i: (0, 0)),
        ],
        out_specs=[
            pl.BlockSpec((1, T), lambda i: (0, 0)),
            pl.BlockSpec((1, 128), lambda i: (0, 0)),
            pl.BlockSpec((1, 128), lambda i: (0, 0)),
            pl.BlockSpec((1, E), lambda i: (0, 0)),
        ],
        out_shape=[
            jax.ShapeDtypeStruct((1, T), jnp.int32),
            jax.ShapeDtypeStruct((1, 128), jnp.int32),
            jax.ShapeDtypeStruct((1, 128), jnp.int32),
            jax.ShapeDtypeStruct((1, E), jnp.float32),
        ],
        scratch_shapes=[
            pltpu.VMEM((1, E), jnp.float32),
            pltpu.VMEM((NGB, GBLK), jnp.int32),
            pltpu.VMEM((NGB, GBLK), jnp.int32),
        ],
    )(xf, gate_W, gate_b)


HALF = CHUNK // 2


def _dispatch_body(pos_hbm, x_hbm, xs_hbm, pos_v0, pos_v1, rows0, rows1,
                   sem_a, sem_b):
    wid = lax.axis_index("s") * NC + lax.axis_index("c")
    base = wid * CHUNK
    pltpu.sync_copy(pos_hbm.at[0, pl.ds(base, HALF)], pos_v0)
    pltpu.sync_copy(pos_hbm.at[0, pl.ds(base + HALF, HALF)], pos_v1)
    in0 = pltpu.async_copy(x_hbm.at[0, pl.ds(base, HALF)], rows0, sem_a)
    in1 = pltpu.async_copy(x_hbm.at[0, pl.ds(base + HALF, HALF)], rows1,
                           sem_b)
    in0.wait()
    out0 = pltpu.async_copy(rows0, xs_hbm.at[pos_v0], sem_a)
    in1.wait()
    out1 = pltpu.async_copy(rows1, xs_hbm.at[pos_v1], sem_b)
    out0.wait()
    out1.wait()


def _dispatch_call(pos, xf):
    mesh = plsc.VectorSubcoreMesh(core_axis_name="c", subcore_axis_name="s")
    fn = functools.partial(
        pl.kernel,
        mesh=mesh,
        out_type=jax.ShapeDtypeStruct((PADDED, D), jnp.float32),
        scratch_types=[
            pltpu.VMEM((HALF,), jnp.int32),
            pltpu.VMEM((HALF,), jnp.int32),
            pltpu.VMEM((HALF, D), jnp.float32),
            pltpu.VMEM((HALF, D), jnp.float32),
            pltpu.SemaphoreType.DMA,
            pltpu.SemaphoreType.DMA,
        ],
    )(_dispatch_body)
    return fn(pos, xf)


def _ffn_body(te_ref, tv_ref, xs_ref, w1_ref, b1_ref, w2_ref, b2_ref,
              pw_ref, pb_ref, out_ref):
    j = pl.program_id(0)

    @pl.when(tv_ref[0, j] == 1)
    def _():
        e = te_ref[0, j]
        xb = xs_ref[...].astype(jnp.bfloat16)
        h = jnp.dot(xb, w1_ref[0].astype(jnp.bfloat16),
                    preferred_element_type=jnp.float32) + b1_ref[pl.ds(e, 1)]
        h = jnp.maximum(h, 0.0).astype(jnp.bfloat16)
        y = jnp.dot(h, w2_ref[0].astype(jnp.bfloat16),
                    preferred_element_type=jnp.float32) + b2_ref[pl.ds(e, 1)]
        out_ref[...] = jnp.dot(
            y.astype(jnp.bfloat16), pw_ref[...].astype(jnp.bfloat16),
            preferred_element_type=jnp.float32) + pb_ref[...]


def _ffn_call(te, tv, xs, W1, b1, W2, b2, proj_W, proj_b2d):
    grid_spec = pltpu.PrefetchScalarGridSpec(
        num_scalar_prefetch=2,
        grid=(MAX_TILES,),
        in_specs=[
            pl.BlockSpec((BLK, D), lambda j, te, tv: (j, 0)),
            pl.BlockSpec((1, D, H), lambda j, te, tv: (te[0, j], 0, 0)),
            pl.BlockSpec((E, H), lambda j, te, tv: (0, 0)),
            pl.BlockSpec((1, H, D), lambda j, te, tv: (te[0, j], 0, 0)),
            pl.BlockSpec((E, D), lambda j, te, tv: (0, 0)),
            pl.BlockSpec((D, D), lambda j, te, tv: (0, 0)),
            pl.BlockSpec((1, D), lambda j, te, tv: (0, 0)),
        ],
        out_specs=pl.BlockSpec((BLK, D), lambda j, te, tv: (j, 0)),
    )
    return pl.pallas_call(
        _ffn_body,
        grid_spec=grid_spec,
        out_shape=jax.ShapeDtypeStruct((PADDED, D), jnp.float32),
    )(te, tv, xs, W1, b1, W2, b2, proj_W, proj_b2d)


def _combine_body(pos_hbm, ys_hbm, out_hbm, pos_v0, pos_v1, rows0, rows1,
                  sem_a, sem_b):
    wid = lax.axis_index("s") * NC + lax.axis_index("c")
    base = wid * CHUNK
    pltpu.sync_copy(pos_hbm.at[0, pl.ds(base, HALF)], pos_v0)
    pltpu.sync_copy(pos_hbm.at[0, pl.ds(base + HALF, HALF)], pos_v1)
    g0 = pltpu.async_copy(ys_hbm.at[pos_v0], rows0, sem_a)
    g1 = pltpu.async_copy(ys_hbm.at[pos_v1], rows1, sem_b)
    g0.wait()
    o0 = pltpu.async_copy(rows0, out_hbm.at[0, pl.ds(base, HALF)], sem_a)
    g1.wait()
    o1 = pltpu.async_copy(rows1, out_hbm.at[0, pl.ds(base + HALF, HALF)],
                          sem_b)
    o0.wait()
    o1.wait()


def _combine_call(pos, ys):
    mesh = plsc.VectorSubcoreMesh(core_axis_name="c", subcore_axis_name="s")
    fn = functools.partial(
        pl.kernel,
        mesh=mesh,
        out_type=jax.ShapeDtypeStruct((1, T, D), jnp.float32),
        scratch_types=[
            pltpu.VMEM((HALF,), jnp.int32),
            pltpu.VMEM((HALF,), jnp.int32),
            pltpu.VMEM((HALF, D), jnp.float32),
            pltpu.VMEM((HALF, D), jnp.float32),
            pltpu.SemaphoreType.DMA,
            pltpu.SemaphoreType.DMA,
        ],
    )(_combine_body)
    return fn(pos, ys)


def kernel(x, gate_W, gate_b, W1, b1, W2, b2, proj_W, proj_b):
    pos, te, tv, loss_o = _gate_call(x, gate_W, gate_b.reshape(1, E))

    xs = _dispatch_call(pos, x)
    ys = _ffn_call(te, tv, xs, W1, b1, W2, b2, proj_W,
                   proj_b.reshape(1, D))
    out = _combine_call(pos, ys)

    loss = loss_o[0, 0].reshape(())
    return out, loss


# speedup vs baseline: 1.0083x; 1.0083x over previous
import functools

import jax
import jax.numpy as jnp
from jax import lax
from jax.experimental import pallas as pl
from jax.experimental.pallas import tpu as pltpu
from jax.experimental.pallas import tpu_sc as plsc

E = 8
D = 768
H = 2048
T = 2048
COEF = 0.01

GBLK = 512
NGB = T // GBLK
BLK = 256
MAX_TILES = T // BLK + E - 1
PADDED = MAX_TILES * BLK
NC, NS = 2, 16
NW = NC * NS
CHUNK = T // NW


def _gate_body(x_ref, gw_ref, gb_ref, pos_ref, te_ref, tv_ref, loss_ref,
               carry_ref, choice_s, poswi_s):
    i = pl.program_id(0)

    @pl.when(i == 0)
    def _():
        carry_ref[...] = jnp.zeros((1, E), jnp.float32)

    @pl.when(i < NGB)
    def _():
        xb = x_ref[0]
        logits = jnp.dot(xb, gw_ref[...],
                         preferred_element_type=jnp.float32) + gb_ref[...]
        lane = lax.broadcasted_iota(jnp.int32, (GBLK, E), 1)
        mx = jnp.max(logits, axis=1, keepdims=True)
        ex = jnp.exp(logits - mx)
        probs = ex / jnp.sum(ex, axis=1, keepdims=True)
        pmax = jnp.max(probs, axis=1, keepdims=True)
        first = jnp.where(probs == pmax, lane, E)
        choice = jnp.min(first, axis=1)

        onehot = (lane == choice[:, None]).astype(jnp.float32)
        r = lax.broadcasted_iota(jnp.int32, (GBLK, GBLK), 0)
        c = lax.broadcasted_iota(jnp.int32, (GBLK, GBLK), 1)
        tri = (c < r).astype(jnp.float32)
        rank = jnp.dot(tri, onehot,
                       preferred_element_type=jnp.float32) + carry_ref[...]
        poswi = jnp.sum(onehot * rank, axis=1)

        choice_s[pl.ds(i, 1), :] = choice.reshape(1, GBLK)
        poswi_s[pl.ds(i, 1), :] = poswi.astype(jnp.int32).reshape(1, GBLK)
        carry_ref[...] = carry_ref[...] + jnp.sum(onehot, axis=0,
                                                  keepdims=True)

    @pl.when(i == NGB)
    def _():
        totals = carry_ref[...]
        p = totals / float(T)
        loss = -jnp.sum(p * jnp.log(p + 1e-10)) * COEF
        loss_ref[...] = jnp.full((1, E), loss, jnp.float32)
        ch = choice_s[...]
        pw = poswi_s[...]
        jb = lax.broadcasted_iota(jnp.int32, (1, 128), 1) * BLK
        acc = jnp.zeros((NGB, GBLK), jnp.int32)
        te = jnp.zeros((1, 128), jnp.int32)
        off = jnp.int32(0)
        for e in range(E):
            te = te + (off <= jb).astype(jnp.int32)
            acc = jnp.where(ch == e, off + pw, acc)
            cnt = totals[0, e].astype(jnp.int32)
            off = off + ((cnt + BLK - 1) // BLK) * BLK
        pos_ref[...] = acc.reshape(1, T)
        te_ref[...] = te - 1
        tv_ref[...] = (jb < off).astype(jnp.int32)


def _gate_call(xf, gate_W, gate_b):
    return pl.pallas_call(
        _gate_body,
        grid=(NGB + 1,),
        in_specs=[
            pl.BlockSpec((1, GBLK, D),
                         lambda i: (0, jnp.minimum(i, NGB - 1), 0)),
            pl.BlockSpec((D, E), lambda i: (0, 0)),
            pl.BlockSpec((1, E), lambda i: (0, 0)),
        ],
        out_specs=[
            pl.BlockSpec((1, T), lambda i: (0, 0)),
            pl.BlockSpec((1, 128), lambda i: (0, 0)),
            pl.BlockSpec((1, 128), lambda i: (0, 0)),
            pl.BlockSpec((1, E), lambda i: (0, 0)),
        ],
        out_shape=[
            jax.ShapeDtypeStruct((1, T), jnp.int32),
            jax.ShapeDtypeStruct((1, 128), jnp.int32),
            jax.ShapeDtypeStruct((1, 128), jnp.int32),
            jax.ShapeDtypeStruct((1, E), jnp.float32),
        ],
        scratch_shapes=[
            pltpu.VMEM((1, E), jnp.float32),
            pltpu.VMEM((NGB, GBLK), jnp.int32),
            pltpu.VMEM((NGB, GBLK), jnp.int32),
        ],
    )(xf, gate_W, gate_b)


def _dispatch_body(pos_hbm, x_hbm, xs_hbm, pos_v, rows_v, sem):
    wid = lax.axis_index("s") * NC + lax.axis_index("c")
    base = wid * CHUNK
    pltpu.sync_copy(pos_hbm.at[0, pl.ds(base, CHUNK)], pos_v)
    pltpu.sync_copy(x_hbm.at[0, pl.ds(base, CHUNK)], rows_v)
    pltpu.async_copy(rows_v, xs_hbm.at[pos_v], sem).wait()


def _dispatch_call(pos, xf):
    mesh = plsc.VectorSubcoreMesh(core_axis_name="c", subcore_axis_name="s")
    fn = functools.partial(
        pl.kernel,
        mesh=mesh,
        out_type=jax.ShapeDtypeStruct((PADDED, D), jnp.float32),
        scratch_types=[
            pltpu.VMEM((CHUNK,), jnp.int32),
            pltpu.VMEM((CHUNK, D), jnp.float32),
            pltpu.SemaphoreType.DMA,
        ],
    )(_dispatch_body)
    return fn(pos, xf)


def _ffn_body(te_ref, tv_ref, xs_ref, w1_ref, b1_ref, w2_ref, b2_ref,
              pw_ref, pb_ref, out_ref):
    j = pl.program_id(0)

    @pl.when(tv_ref[0, j] == 1)
    def _():
        e = te_ref[0, j]
        xb = xs_ref[...].astype(jnp.bfloat16)
        h = jnp.dot(xb, w1_ref[0].astype(jnp.bfloat16),
                    preferred_element_type=jnp.float32) + b1_ref[pl.ds(e, 1)]
        h = jnp.maximum(h, 0.0).astype(jnp.bfloat16)
        y = jnp.dot(h, w2_ref[0].astype(jnp.bfloat16),
                    preferred_element_type=jnp.float32) + b2_ref[pl.ds(e, 1)]
        out_ref[...] = jnp.dot(
            y.astype(jnp.bfloat16), pw_ref[...].astype(jnp.bfloat16),
            preferred_element_type=jnp.float32) + pb_ref[...]


def _ffn_call(te, tv, xs, W1, b1, W2, b2, proj_W, proj_b2d):
    grid_spec = pltpu.PrefetchScalarGridSpec(
        num_scalar_prefetch=2,
        grid=(MAX_TILES,),
        in_specs=[
            pl.BlockSpec((BLK, D), lambda j, te, tv: (j, 0)),
            pl.BlockSpec((1, D, H), lambda j, te, tv: (te[0, j], 0, 0)),
            pl.BlockSpec((E, H), lambda j, te, tv: (0, 0)),
            pl.BlockSpec((1, H, D), lambda j, te, tv: (te[0, j], 0, 0)),
            pl.BlockSpec((E, D), lambda j, te, tv: (0, 0)),
            pl.BlockSpec((D, D), lambda j, te, tv: (0, 0)),
            pl.BlockSpec((1, D), lambda j, te, tv: (0, 0)),
        ],
        out_specs=pl.BlockSpec((BLK, D), lambda j, te, tv: (j, 0)),
    )
    return pl.pallas_call(
        _ffn_body,
        grid_spec=grid_spec,
        out_shape=jax.ShapeDtypeStruct((PADDED, D), jnp.float32),
    )(te, tv, xs, W1, b1, W2, b2, proj_W, proj_b2d)


def _combine_body(pos_hbm, ys_hbm, out_hbm, pos_v, rows_v, sem):
    wid = lax.axis_index("s") * NC + lax.axis_index("c")
    base = wid * CHUNK
    pltpu.sync_copy(pos_hbm.at[0, pl.ds(base, CHUNK)], pos_v)
    pltpu.async_copy(ys_hbm.at[pos_v], rows_v, sem).wait()
    pltpu.sync_copy(rows_v, out_hbm.at[0, pl.ds(base, CHUNK)])


def _combine_call(pos, ys):
    mesh = plsc.VectorSubcoreMesh(core_axis_name="c", subcore_axis_name="s")
    fn = functools.partial(
        pl.kernel,
        mesh=mesh,
        out_type=jax.ShapeDtypeStruct((1, T, D), jnp.float32),
        scratch_types=[
            pltpu.VMEM((CHUNK,), jnp.int32),
            pltpu.VMEM((CHUNK, D), jnp.float32),
            pltpu.SemaphoreType.DMA,
        ],
    )(_combine_body)
    return fn(pos, ys)


def kernel(x, gate_W, gate_b, W1, b1, W2, b2, proj_W, proj_b):
    pos, te, tv, loss_o = _gate_call(x, gate_W, gate_b.reshape(1, E))

    xs = _dispatch_call(pos, x)
    ys = _ffn_call(te, tv, xs, W1, b1, W2, b2, proj_W,
                   proj_b.reshape(1, D))
    out = _combine_call(pos, ys)

    loss = loss_o[0, 0].reshape(())
    return out, loss
